# baseline (speedup 1.0000x reference)
import jax
import jax.numpy as jnp
from jax.experimental import pallas as pl
from jax.experimental.pallas import tpu as pltpu
from jax.experimental.pallas import tpu_sc as plsc

M, B, D = 65536, 8192, 256
T = M + B

BLK = 4096
NM = M // BLK
NB = B // BLK


def _tc_body(k_ref, nk_ref, ok_ref):
    i = pl.program_id(0)

    @pl.when(i < NM)
    def _():
        ok_ref[...] = k_ref[...]

    @pl.when(i >= NM)
    def _():
        ok_ref[...] = nk_ref[...]


def _tc_concat(keys, new_keys):
    return pl.pallas_call(
        _tc_body,
        grid=(NM + NB,),
        in_specs=[
            pl.BlockSpec((BLK, D), lambda i: (jnp.minimum(i, NM - 1), 0)),
            pl.BlockSpec((BLK, D), lambda i: (jnp.maximum(i - NM, 0), 0)),
        ],
        out_specs=pl.BlockSpec((BLK, D), lambda i: (i, 0)),
        out_shape=jax.ShapeDtypeStruct((T, D), keys.dtype),
    )(keys, new_keys)


K_TC = 49152
NW = 32
CH = 128
NBUF = 3
W_OLD = (M - K_TC) // NW
W_NEW = B // NW
N_OLD = W_OLD // CH
N_NEW = W_NEW // CH


def _sc_tail(keys, new_keys):
    mesh = plsc.VectorSubcoreMesh(
        core_axis_name="core", subcore_axis_name="subcore")

    @pl.kernel(
        out_type=jax.ShapeDtypeStruct((T, D), keys.dtype),
        mesh=mesh,
        scratch_types=[
            [pltpu.VMEM((CH, D), keys.dtype) for _ in range(NBUF)],
            pltpu.SemaphoreType.DMA((NBUF,)),
            pltpu.SemaphoreType.DMA((NBUF,)),
        ],
    )
    def sc_copy(k_hbm, nk_hbm, ok_hbm, bufs, in_sems, out_sems):
        core = jax.lax.axis_index("core")
        sub = jax.lax.axis_index("subcore")
        wid = sub * 2 + core
        old_base = K_TC + wid * W_OLD
        new_base = wid * W_NEW

        chunks = []
        for c in range(N_OLD):
            off = old_base + c * CH
            chunks.append((k_hbm, off, off))
        for c in range(N_NEW):
            off = new_base + c * CH
            chunks.append((nk_hbm, off, M + off))
        n = len(chunks)

        in_cp = [None] * n
        out_cp = [None] * n

        def start_in(i):
            src, soff, _ = chunks[i]
            b = i % NBUF
            in_cp[i] = pltpu.make_async_copy(
                src.at[pl.ds(soff, CH), :], bufs[b], in_sems.at[b])
            in_cp[i].start()

        def start_out(i):
            _, _, doff = chunks[i]
            b = i % NBUF
            out_cp[i] = pltpu.make_async_copy(
                bufs[b], ok_hbm.at[pl.ds(doff, CH), :], out_sems.at[b])
            out_cp[i].start()

        for i in range(min(NBUF, n)):
            start_in(i)
        for i in range(n):
            in_cp[i].wait()
            start_out(i)
            j = i + NBUF - 1
            if i >= 1 and j < n:
                out_cp[i - 1].wait()
                start_in(j)
        for i in range(max(0, n - NBUF + 1), n):
            out_cp[i].wait()

    return sc_copy(keys, new_keys)


NFILL = K_TC // BLK


def _fill_body(k_ref, alias_ref, ok_ref):
    ok_ref[...] = k_ref[...]


def _tc_fill(keys, partial):
    return pl.pallas_call(
        _fill_body,
        grid=(NFILL,),
        in_specs=[
            pl.BlockSpec((BLK, D), lambda i: (i, 0)),
            pl.BlockSpec(memory_space=pltpu.MemorySpace.HBM),
        ],
        out_specs=pl.BlockSpec((BLK, D), lambda i: (i, 0)),
        out_shape=jax.ShapeDtypeStruct((T, D), keys.dtype),
        input_output_aliases={1: 0},
    )(keys, partial)


def kernel(keys, vals, new_keys, new_vals):
    partial_keys = _sc_tail(keys, new_keys)
    out_vals = _tc_concat(vals, new_vals)
    out_keys = _tc_fill(keys, partial_keys)
    return (out_keys, out_vals)


# speedup vs baseline: 1.2264x; 1.2264x over previous
import jax
import jax.numpy as jnp
from jax.experimental import pallas as pl
from jax.experimental.pallas import tpu as pltpu

M, B, D = 65536, 8192, 256
T = M + B
CH = 4096
NBUF = 6
N_OLD = M // CH
N_NEW = B // CH


def _dma_body(k, v, nk, nv, ok, ov, *rest):
    bufs = rest[:NBUF]
    in_sems, out_sems = rest[NBUF], rest[NBUF + 1]

    chunks = []
    for c in range(N_OLD):
        chunks.append((k, c * CH, ok, c * CH))
        chunks.append((v, c * CH, ov, c * CH))
    for c in range(N_NEW):
        chunks.append((nk, c * CH, ok, M + c * CH))
        chunks.append((nv, c * CH, ov, M + c * CH))
    n = len(chunks)

    in_cp = [None] * n
    out_cp = [None] * n

    def start_in(i):
        src, soff, _, _ = chunks[i]
        b = i % NBUF
        in_cp[i] = pltpu.make_async_copy(
            src.at[pl.ds(soff, CH), :], bufs[b], in_sems.at[b])
        in_cp[i].start()

    def start_out(i):
        _, _, dst, doff = chunks[i]
        b = i % NBUF
        out_cp[i] = pltpu.make_async_copy(
            bufs[b], dst.at[pl.ds(doff, CH), :], out_sems.at[b])
        out_cp[i].start()

    for i in range(min(NBUF, n)):
        start_in(i)
    for i in range(n):
        in_cp[i].wait()
        start_out(i)
        j = i + NBUF - 1
        if i >= 1 and j < n:
            out_cp[i - 1].wait()
            start_in(j)
    for i in range(max(0, n - NBUF), n):
        out_cp[i].wait()


def kernel(keys, vals, new_keys, new_vals):
    hbm = pl.BlockSpec(memory_space=pltpu.MemorySpace.HBM)
    out_shape = jax.ShapeDtypeStruct((T, D), keys.dtype)
    scratch = [pltpu.VMEM((CH, D), keys.dtype) for _ in range(NBUF)]
    scratch += [pltpu.SemaphoreType.DMA((NBUF,)),
                pltpu.SemaphoreType.DMA((NBUF,))]
    return pl.pallas_call(
        _dma_body,
        in_specs=[hbm, hbm, hbm, hbm],
        out_specs=[hbm, hbm],
        out_shape=[out_shape, out_shape],
        scratch_shapes=scratch,
    )(keys, vals, new_keys, new_vals)


# speedup vs baseline: 1.2286x; 1.0018x over previous
import jax
import jax.numpy as jnp
from jax.experimental import pallas as pl
from jax.experimental.pallas import tpu as pltpu

M, B, D = 65536, 8192, 256
T = M + B
CH = 4096
NBUF = 10
N_OLD = M // CH
N_NEW = B // CH


def _dma_body(k, v, nk, nv, ok, ov, *rest):
    bufs = rest[:NBUF]
    in_sems, out_sems = rest[NBUF], rest[NBUF + 1]

    chunks = []
    for c in range(N_OLD):
        chunks.append((k, c * CH, ok, c * CH))
        chunks.append((v, c * CH, ov, c * CH))
    for c in range(N_NEW):
        chunks.append((nk, c * CH, ok, M + c * CH))
        chunks.append((nv, c * CH, ov, M + c * CH))
    n = len(chunks)

    in_cp = [None] * n
    out_cp = [None] * n

    def start_in(i):
        src, soff, _, _ = chunks[i]
        b = i % NBUF
        in_cp[i] = pltpu.make_async_copy(
            src.at[pl.ds(soff, CH), :], bufs[b], in_sems.at[b])
        in_cp[i].start()

    def start_out(i):
        _, _, dst, doff = chunks[i]
        b = i % NBUF
        out_cp[i] = pltpu.make_async_copy(
            bufs[b], dst.at[pl.ds(doff, CH), :], out_sems.at[b])
        out_cp[i].start()

    for i in range(min(NBUF, n)):
        start_in(i)
    for i in range(n):
        in_cp[i].wait()
        start_out(i)
        j = i + NBUF - 1
        if i >= 1 and j < n:
            out_cp[i - 1].wait()
            start_in(j)
    for i in range(max(0, n - NBUF), n):
        out_cp[i].wait()


def kernel(keys, vals, new_keys, new_vals):
    hbm = pl.BlockSpec(memory_space=pltpu.MemorySpace.HBM)
    out_shape = jax.ShapeDtypeStruct((T, D), keys.dtype)
    scratch = [pltpu.VMEM((CH, D), keys.dtype) for _ in range(NBUF)]
    scratch += [pltpu.SemaphoreType.DMA((NBUF,)),
                pltpu.SemaphoreType.DMA((NBUF,))]
    return pl.pallas_call(
        _dma_body,
        in_specs=[hbm, hbm, hbm, hbm],
        out_specs=[hbm, hbm],
        out_shape=[out_shape, out_shape],
        scratch_shapes=scratch,
    )(keys, vals, new_keys, new_vals)


# speedup vs baseline: 1.2310x; 1.0019x over previous
import jax
import jax.numpy as jnp
from jax.experimental import pallas as pl
from jax.experimental.pallas import tpu as pltpu

M, B, D = 65536, 8192, 256
T = M + B
CH = 8192
NBUF = 6
N_OLD = M // CH
N_NEW = B // CH


def _dma_body(k, v, nk, nv, ok, ov, *rest):
    bufs = rest[:NBUF]
    in_sems, out_sems = rest[NBUF], rest[NBUF + 1]

    chunks = []
    for c in range(N_OLD):
        chunks.append((k, c * CH, ok, c * CH))
        chunks.append((v, c * CH, ov, c * CH))
    for c in range(N_NEW):
        chunks.append((nk, c * CH, ok, M + c * CH))
        chunks.append((nv, c * CH, ov, M + c * CH))
    n = len(chunks)

    in_cp = [None] * n
    out_cp = [None] * n

    def start_in(i):
        src, soff, _, _ = chunks[i]
        b = i % NBUF
        in_cp[i] = pltpu.make_async_copy(
            src.at[pl.ds(soff, CH), :], bufs[b], in_sems.at[b])
        in_cp[i].start()

    def start_out(i):
        _, _, dst, doff = chunks[i]
        b = i % NBUF
        out_cp[i] = pltpu.make_async_copy(
            bufs[b], dst.at[pl.ds(doff, CH), :], out_sems.at[b])
        out_cp[i].start()

    for i in range(min(NBUF, n)):
        start_in(i)
    for i in range(n):
        in_cp[i].wait()
        start_out(i)
        j = i + NBUF - 1
        if i >= 1 and j < n:
            out_cp[i - 1].wait()
            start_in(j)
    for i in range(max(0, n - NBUF), n):
        out_cp[i].wait()


def kernel(keys, vals, new_keys, new_vals):
    hbm = pl.BlockSpec(memory_space=pltpu.MemorySpace.HBM)
    out_shape = jax.ShapeDtypeStruct((T, D), keys.dtype)
    scratch = [pltpu.VMEM((CH, D), keys.dtype) for _ in range(NBUF)]
    scratch += [pltpu.SemaphoreType.DMA((NBUF,)),
                pltpu.SemaphoreType.DMA((NBUF,))]
    return pl.pallas_call(
        _dma_body,
        in_specs=[hbm, hbm, hbm, hbm],
        out_specs=[hbm, hbm],
        out_shape=[out_shape, out_shape],
        scratch_shapes=scratch,
    )(keys, vals, new_keys, new_vals)
